# baseline (speedup 1.0000x reference)
import functools

import numpy as np
import jax
import jax.numpy as jnp
from jax import lax
from jax.experimental import pallas as pl
from jax.experimental.pallas import tpu as pltpu
from jax.experimental.pallas import tpu_sc as plsc
from jax._src.pallas.primitives import reciprocal as _pl_reciprocal

_H = 32
_W = 32
_C = 196
_HW = _H * _W
_NB = 64
_ROWS = _NB * _C
_LANES = 128
_GRID = _ROWS // _LANES
_NCHUNK = _HW // 8
_DEPTH_N = _NB * 640 * 480

_ORDER = [4 * i + c for c in range(4) for i in range(32)]

_ZPX = float(np.float32(640.0) / np.float32(900.0))
_ZPY = float(np.float32(480.0) / np.float32(900.0))

_pxg, _pyg = np.meshgrid(np.linspace(-1.0, 1.0, _H), np.linspace(-1.0, 1.0, _W))
_POS_X_NP = _pxg.reshape(_HW, 1).astype(np.float32)
_POS_Y_NP = _pyg.reshape(_HW, 1).astype(np.float32)


def _fold8(a, op):
    t1 = op(a[0:4, :], a[4:8, :])
    t2 = op(t1[0:2, :], t1[2:4, :])
    return op(t2[0:1, :], t2[1:2, :])


def _make_tc_body(order, div_mode):
    def body(ft_ref, px_ref, py_ref, ex_ref, ey_ref, ci_ref, e_ref):
        j = pl.program_id(0)
        m = ft_ref[pl.ds(0, 8), :]
        for t in range(1, _NCHUNK):
            m = jnp.maximum(m, ft_ref[pl.ds(8 * t, 8), :])
        m1 = _fold8(m, jnp.maximum)

        acc = None
        for t in order:
            x = ft_ref[pl.ds(8 * t, 8), :]
            et = jnp.exp(x - m1)
            e_ref[pl.ds(8 * t, 8), :] = et
            acc = et if acc is None else acc + et
        s = _fold8(acc, jnp.add)

        if div_mode == "approx":
            r = _pl_reciprocal(s, approx=True)
        elif div_mode == "recip_once":
            r = 1.0 / s
        ex = ey = None
        for g in range(4):
            accx = accy = None
            for t in range(32 * g, 32 * g + 32):
                et = e_ref[pl.ds(8 * t, 8), :]
                if div_mode == "divide":
                    w = et / s
                else:
                    w = et * r
                tx = px_ref[pl.ds(8 * t, 8), :] * w
                ty = py_ref[pl.ds(8 * t, 8), :] * w
                accx = tx if accx is None else accx + tx
                accy = ty if accy is None else accy + ty
            fx = _fold8(accx, jnp.add)
            fy = _fold8(accy, jnp.add)
            ex = fx if ex is None else ex + fx
            ey = fy if ey is None else ey + fy

        rows = j * _LANES + lax.broadcasted_iota(jnp.int32, (1, _LANES), 1)
        bf = jnp.floor((rows.astype(jnp.float32) + 0.5) * (1.0 / 196.0))
        coord = (ex * 320.0 + 319.0) + ((ey * 240.0 + 239.0) * 640.0)
        tot = coord + bf * 307200.0
        ci = jnp.round(tot).astype(jnp.int32)
        ci = jnp.where(ci < 0, ci + _DEPTH_N, ci)
        ci = jnp.clip(ci, 0, _DEPTH_N - 1)
        ex_ref[...] = ex.reshape(1, 1, _LANES)
        ey_ref[...] = ey.reshape(1, 1, _LANES)
        ci_ref[...] = ci.reshape(1, 1, _LANES)
    return body


def _tc_stage(ft, px, py, order=tuple(_ORDER), div_mode="approx"):
    return pl.pallas_call(
        _make_tc_body(list(order), div_mode),
        grid=(_GRID,),
        in_specs=[
            pl.BlockSpec((_HW, _LANES), lambda i: (0, i)),
            pl.BlockSpec((_HW, 1), lambda i: (0, 0)),
            pl.BlockSpec((_HW, 1), lambda i: (0, 0)),
        ],
        out_specs=[
            pl.BlockSpec((1, 1, _LANES), lambda i: (i, 0, 0)),
            pl.BlockSpec((1, 1, _LANES), lambda i: (i, 0, 0)),
            pl.BlockSpec((1, 1, _LANES), lambda i: (i, 0, 0)),
        ],
        out_shape=[
            jax.ShapeDtypeStruct((_GRID, 1, _LANES), jnp.float32),
            jax.ShapeDtypeStruct((_GRID, 1, _LANES), jnp.float32),
            jax.ShapeDtypeStruct((_GRID, 1, _LANES), jnp.int32),
        ],
        scratch_shapes=[pltpu.VMEM((_HW, _LANES), jnp.float32)],
    )(ft, px, py)


_NC = 2
_NS = 16
_NW = _NC * _NS
_BPW = _ROWS // _NW


@functools.cache
def _make_sc_stage():
    return functools.partial(
        pl.kernel,
        mesh=plsc.VectorSubcoreMesh(core_axis_name="c", subcore_axis_name="s"),
        out_type=[
            jax.ShapeDtypeStruct((_ROWS,), jnp.float32),
            jax.ShapeDtypeStruct((_ROWS,), jnp.float32),
            jax.ShapeDtypeStruct((_ROWS,), jnp.float32),
        ],
        scratch_types=[
            pltpu.VMEM((_BPW,), jnp.float32),
            pltpu.VMEM((_BPW,), jnp.float32),
            pltpu.VMEM((_BPW,), jnp.int32),
            pltpu.VMEM((_BPW,), jnp.float32),
            pltpu.VMEM((_BPW,), jnp.float32),
            pltpu.VMEM((_BPW,), jnp.float32),
            pltpu.SemaphoreType.DMA,
        ],
    )(_sc_body)


def _sc_body(ex_hbm, ey_hbm, ci_hbm, depth_hbm, xz_hbm, yz_hbm, z_hbm,
             ex_v, ey_v, ci_v, z_v, xz_v, yz_v, sem):
    wid = lax.axis_index("s") * _NC + lax.axis_index("c")
    base = wid * _BPW
    pltpu.sync_copy(ex_hbm.at[pl.ds(base, _BPW)], ex_v)
    pltpu.sync_copy(ey_hbm.at[pl.ds(base, _BPW)], ey_v)
    pltpu.sync_copy(ci_hbm.at[pl.ds(base, _BPW)], ci_v)
    handles = []
    for off, n in ((0, 128), (128, 128), (256, 128), (384, 8)):
        handles.append(
            pltpu.async_copy(
                depth_hbm.at[ci_v.at[pl.ds(off, n)]],
                z_v.at[pl.ds(off, n)],
                sem,
            )
        )
    for h in handles:
        h.wait()
    for off in [16 * j for j in range(24)] + [_BPW - 16]:
        z = z_v[pl.ds(off, 16)]
        ex16 = ex_v[pl.ds(off, 16)]
        ey16 = ey_v[pl.ds(off, 16)]
        xz_v[pl.ds(off, 16)] = ex16 * (z * _ZPX)
        yz_v[pl.ds(off, 16)] = ey16 * (z * _ZPY)
    pltpu.sync_copy(xz_v, xz_hbm.at[pl.ds(base, _BPW)])
    pltpu.sync_copy(yz_v, yz_hbm.at[pl.ds(base, _BPW)])
    pltpu.sync_copy(z_v, z_hbm.at[pl.ds(base, _BPW)])


def kernel(feature, depth):
    ft = feature.reshape(_ROWS, _HW).T
    px = jnp.asarray(_POS_X_NP)
    py = jnp.asarray(_POS_Y_NP)
    ex, ey, ci = _tc_stage(ft, px, py)
    xz, yz, z = _make_sc_stage()(
        ex.reshape(_ROWS),
        ey.reshape(_ROWS),
        ci.reshape(_ROWS),
        depth.reshape(_DEPTH_N),
    )
    out = jnp.stack([xz, yz, z], axis=1)
    return out.reshape(_NB, _C * 3)


# speedup vs baseline: 1.0014x; 1.0014x over previous
import functools

import numpy as np
import jax
import jax.numpy as jnp
from jax import lax
from jax.experimental import pallas as pl
from jax.experimental.pallas import tpu as pltpu
from jax.experimental.pallas import tpu_sc as plsc
from jax._src.pallas.primitives import reciprocal as _pl_reciprocal

_H = 32
_W = 32
_C = 196
_HW = _H * _W
_NB = 64
_ROWS = _NB * _C
_LANES = 128
_GRID = _ROWS // _LANES
_NCHUNK = _HW // 8
_DEPTH_N = _NB * 640 * 480

_ORDER = [4 * i + c for c in range(4) for i in range(32)]

_ZPX = float(np.float32(640.0) / np.float32(900.0))
_ZPY = float(np.float32(480.0) / np.float32(900.0))

_pxg, _pyg = np.meshgrid(np.linspace(-1.0, 1.0, _H), np.linspace(-1.0, 1.0, _W))
_POS_X_NP = _pxg.reshape(_HW, 1).astype(np.float32)
_POS_Y_NP = _pyg.reshape(_HW, 1).astype(np.float32)


def _fold8(a, op):
    t1 = op(a[0:4, :], a[4:8, :])
    t2 = op(t1[0:2, :], t1[2:4, :])
    return op(t2[0:1, :], t2[1:2, :])


def _make_tc_body(order=tuple(_ORDER)):
    def body(ft_ref, px_ref, py_ref, ex_ref, ey_ref, ci_ref, e_ref):
        j = pl.program_id(0)
        m = ft_ref[pl.ds(0, 8), :]
        for t in range(1, _NCHUNK):
            m = jnp.maximum(m, ft_ref[pl.ds(8 * t, 8), :])
        m1 = _fold8(m, jnp.maximum)

        acc = None
        for t in order:
            x = ft_ref[pl.ds(8 * t, 8), :]
            et = jnp.exp(x - m1)
            e_ref[pl.ds(8 * t, 8), :] = et
            acc = et if acc is None else acc + et
        s = _fold8(acc, jnp.add)

        r = _pl_reciprocal(s, approx=True)
        ex = ey = None
        for g in range(4):
            accx = accy = None
            for t in range(32 * g, 32 * g + 32):
                et = e_ref[pl.ds(8 * t, 8), :]
                w = et * r
                tx = px_ref[pl.ds(8 * t, 8), :] * w
                ty = py_ref[pl.ds(8 * t, 8), :] * w
                accx = tx if accx is None else accx + tx
                accy = ty if accy is None else accy + ty
            fx = _fold8(accx, jnp.add)
            fy = _fold8(accy, jnp.add)
            ex = fx if ex is None else ex + fx
            ey = fy if ey is None else ey + fy

        rows = j * _LANES + lax.broadcasted_iota(jnp.int32, (1, _LANES), 1)
        bf = jnp.floor((rows.astype(jnp.float32) + 0.5) * (1.0 / 196.0))
        coord = (ex * 320.0 + 319.0) + ((ey * 240.0 + 239.0) * 640.0)
        tot = coord + bf * 307200.0
        ci = jnp.round(tot).astype(jnp.int32)
        ci = jnp.where(ci < 0, ci + _DEPTH_N, ci)
        ci = jnp.clip(ci, 0, _DEPTH_N - 1)
        ex_ref[...] = ex.reshape(1, 1, _LANES)
        ey_ref[...] = ey.reshape(1, 1, _LANES)
        ci_ref[...] = ci.reshape(1, 1, _LANES)
    return body


def _tc_stage(ft, px, py):
    return pl.pallas_call(
        _make_tc_body(),
        grid=(_GRID,),
        in_specs=[
            pl.BlockSpec((_HW, _LANES), lambda i: (0, i)),
            pl.BlockSpec((_HW, 1), lambda i: (0, 0)),
            pl.BlockSpec((_HW, 1), lambda i: (0, 0)),
        ],
        out_specs=[
            pl.BlockSpec((1, 1, _LANES), lambda i: (i, 0, 0)),
            pl.BlockSpec((1, 1, _LANES), lambda i: (i, 0, 0)),
            pl.BlockSpec((1, 1, _LANES), lambda i: (i, 0, 0)),
        ],
        out_shape=[
            jax.ShapeDtypeStruct((_GRID, 1, _LANES), jnp.float32),
            jax.ShapeDtypeStruct((_GRID, 1, _LANES), jnp.float32),
            jax.ShapeDtypeStruct((_GRID, 1, _LANES), jnp.int32),
        ],
        scratch_shapes=[pltpu.VMEM((_HW, _LANES), jnp.float32)],
    )(ft, px, py)


_NC = 2
_NS = 16
_NW = _NC * _NS
_BPW = _ROWS // _NW


@functools.cache
def _make_sc_stage():
    return functools.partial(
        pl.kernel,
        mesh=plsc.VectorSubcoreMesh(core_axis_name="c", subcore_axis_name="s"),
        out_type=[
            jax.ShapeDtypeStruct((_ROWS,), jnp.float32),
            jax.ShapeDtypeStruct((_ROWS,), jnp.float32),
            jax.ShapeDtypeStruct((_ROWS,), jnp.float32),
        ],
        scratch_types=[
            pltpu.VMEM((_BPW,), jnp.float32),
            pltpu.VMEM((_BPW,), jnp.float32),
            pltpu.VMEM((_BPW,), jnp.int32),
            pltpu.VMEM((_BPW,), jnp.float32),
            pltpu.VMEM((_BPW,), jnp.float32),
            pltpu.VMEM((_BPW,), jnp.float32),
            pltpu.SemaphoreType.DMA,
        ],
    )(_sc_body)


def _sc_body(ex_hbm, ey_hbm, ci_hbm, depth_hbm, xz_hbm, yz_hbm, z_hbm,
             ex_v, ey_v, ci_v, z_v, xz_v, yz_v, sem):
    wid = lax.axis_index("s") * _NC + lax.axis_index("c")
    base = wid * _BPW
    pltpu.sync_copy(ex_hbm.at[pl.ds(base, _BPW)], ex_v)
    pltpu.sync_copy(ey_hbm.at[pl.ds(base, _BPW)], ey_v)
    pltpu.sync_copy(ci_hbm.at[pl.ds(base, _BPW)], ci_v)
    handles = []
    for off, n in ((0, 128), (128, 128), (256, 128), (384, 8)):
        handles.append(
            pltpu.async_copy(
                depth_hbm.at[ci_v.at[pl.ds(off, n)]],
                z_v.at[pl.ds(off, n)],
                sem,
            )
        )
    for h in handles:
        h.wait()
    for off in [16 * j for j in range(24)] + [_BPW - 16]:
        z = z_v[pl.ds(off, 16)]
        ex16 = ex_v[pl.ds(off, 16)]
        ey16 = ey_v[pl.ds(off, 16)]
        xz_v[pl.ds(off, 16)] = ex16 * (z * _ZPX)
        yz_v[pl.ds(off, 16)] = ey16 * (z * _ZPY)
    pltpu.sync_copy(xz_v, xz_hbm.at[pl.ds(base, _BPW)])
    pltpu.sync_copy(yz_v, yz_hbm.at[pl.ds(base, _BPW)])
    pltpu.sync_copy(z_v, z_hbm.at[pl.ds(base, _BPW)])


def kernel(feature, depth):
    ft = feature.reshape(_ROWS, _HW).T
    px = jnp.asarray(_POS_X_NP)
    py = jnp.asarray(_POS_Y_NP)
    ex, ey, ci = _tc_stage(ft, px, py)
    xz, yz, z = _make_sc_stage()(
        ex.reshape(_ROWS),
        ey.reshape(_ROWS),
        ci.reshape(_ROWS),
        depth.reshape(_DEPTH_N),
    )
    out = jnp.stack([xz, yz, z], axis=1)
    return out.reshape(_NB, _C * 3)
